# baseline (speedup 1.0000x reference)
import functools

import jax
import jax.numpy as jnp
from jax import lax
from jax.experimental import pallas as pl
from jax.experimental.pallas import tpu as pltpu
from jax.experimental.pallas import tpu_sc as plsc

VOCAB = 1000000
D = 64
B = 4096
L = 200
N = B * L
LANES = 16
CHUNK = 512
EPS = 1e-5


def _positional_encoding(d_model, max_len):
    pos = jnp.arange(max_len, dtype=jnp.float32)[:, None]
    div = jnp.exp(
        jnp.arange(0, d_model, 2, dtype=jnp.float32) * (-jnp.log(10000.0) / d_model)
    )
    ang = pos * div[None, :]
    pe = jnp.zeros((max_len, d_model), dtype=jnp.float32)
    pe = pe.at[:, 0::2].set(jnp.sin(ang))
    pe = pe.at[:, 1::2].set(jnp.cos(ang))
    return pe


def _rsqrt_sc(x):
    i = lax.bitcast_convert_type(x, jnp.int32)
    i = jnp.int32(0x5F3759DF) - (i >> 1)
    y = lax.bitcast_convert_type(i, jnp.float32)
    for _ in range(3):
        y = y * (1.5 - 0.5 * x * y * y)
    return y


def _sc_body(nc, ids_hbm, tags_hbm, table_hbm, comb_hbm, gam_hbm, bet_hbm,
             out_hbm, idx_v, tag_v, rows_v, comb_v, gam_v, bet_v, tbuf_v, sem):
    wid = lax.axis_index("s") * nc + lax.axis_index("c")
    per_w = N // (nc * 16)
    base = wid * per_w
    n_chunks = per_w // CHUNK

    pltpu.sync_copy(comb_hbm, comb_v)
    pltpu.sync_copy(gam_hbm, gam_v)
    pltpu.sync_copy(bet_hbm, bet_v)

    gs, bs = [], []
    for k in range(D // LANES):
        gv = gam_v[pl.ds(k * LANES, LANES)]
        bv = bet_v[pl.ds(k * LANES, LANES)]
        for j in range(LANES):
            gs.append(gv[j])
            bs.append(bv[j])

    lanes = lax.iota(jnp.int32, LANES)

    @pl.loop(0, n_chunks)
    def _chunk(ci):
        off = base + ci * CHUNK
        pltpu.sync_copy(ids_hbm.at[pl.ds(off, CHUNK)], idx_v)
        pltpu.sync_copy(tags_hbm.at[pl.ds(off, CHUNK)], tag_v)
        pltpu.async_copy(table_hbm.at[idx_v], rows_v, sem).wait()

        @pl.loop(0, CHUNK // LANES)
        def _group(g):
            tok = g * LANES + lanes
            gpos = off + tok
            lpos = lax.rem(gpos, jnp.int32(L))
            tag16 = tag_v[pl.ds(g * LANES, LANES)]
            crow = tag16 * L + lpos

            acc = jnp.zeros((LANES,), jnp.float32)
            acc2 = jnp.zeros((LANES,), jnp.float32)
            for d in range(D):
                dvec = jnp.full((LANES,), d, jnp.int32)
                vt = plsc.load_gather(rows_v, [tok, dvec])
                vc = plsc.load_gather(comb_v, [crow, dvec])
                v = vt + vc
                tbuf_v[pl.ds(d * LANES, LANES)] = v
                acc = acc + v
                acc2 = acc2 + v * v

            mean = acc * (1.0 / D)
            var = acc2 * (1.0 / D) - mean * mean
            rstd = _rsqrt_sc(var + EPS)
            mrs = mean * rstd
            for d in range(D):
                dvec = jnp.full((LANES,), d, jnp.int32)
                v = tbuf_v[pl.ds(d * LANES, LANES)]
                outv = (v * rstd - mrs) * gs[d] + bs[d]
                plsc.store_scatter(rows_v, [tok, dvec], outv)

        pltpu.sync_copy(rows_v, out_hbm.at[pl.ds(off, CHUNK)])


def kernel(input_ids, is_buyer_tags, token_table, buyer_table, ln_gamma, ln_beta):
    info = plsc.get_sparse_core_info()
    nc = info.num_cores

    ids_flat = input_ids.reshape(N).astype(jnp.int32)
    tags_flat = is_buyer_tags.reshape(N).astype(jnp.int32)
    pe = _positional_encoding(D, L)
    comb = (pe[None, :, :] + buyer_table[:, None, :]).reshape(2 * L, D)

    mesh = plsc.VectorSubcoreMesh(core_axis_name="c", subcore_axis_name="s")
    run = pl.kernel(
        functools.partial(_sc_body, nc),
        out_type=jax.ShapeDtypeStruct((N, D), jnp.float32),
        mesh=mesh,
        scratch_types=[
            pltpu.VMEM((CHUNK,), jnp.int32),
            pltpu.VMEM((CHUNK,), jnp.int32),
            pltpu.VMEM((CHUNK, D), jnp.float32),
            pltpu.VMEM((2 * L, D), jnp.float32),
            pltpu.VMEM((D,), jnp.float32),
            pltpu.VMEM((D,), jnp.float32),
            pltpu.VMEM((D * LANES,), jnp.float32),
            pltpu.SemaphoreType.DMA,
        ],
        compiler_params=pltpu.CompilerParams(
            use_tc_tiling_on_sc=False, needs_layout_passes=False
        ),
    )
    out = run(ids_flat, tags_flat, token_table, comb, ln_gamma, ln_beta)
    return out.reshape(B, L, D)


# speedup vs baseline: 1.6213x; 1.6213x over previous
import functools

import jax
import jax.numpy as jnp
from jax import lax
from jax.experimental import pallas as pl
from jax.experimental.pallas import tpu as pltpu
from jax.experimental.pallas import tpu_sc as plsc

VOCAB = 1000000
D = 64
B = 4096
L = 200
N = B * L
LANES = 16
CHUNK = 512
EPS = 1e-5


def _positional_encoding(d_model, max_len):
    pos = jnp.arange(max_len, dtype=jnp.float32)[:, None]
    div = jnp.exp(
        jnp.arange(0, d_model, 2, dtype=jnp.float32) * (-jnp.log(10000.0) / d_model)
    )
    ang = pos * div[None, :]
    pe = jnp.zeros((max_len, d_model), dtype=jnp.float32)
    pe = pe.at[:, 0::2].set(jnp.sin(ang))
    pe = pe.at[:, 1::2].set(jnp.cos(ang))
    return pe


def _rsqrt_sc(x):
    i = lax.bitcast_convert_type(x, jnp.int32)
    i = jnp.int32(0x5F3759DF) - (i >> 1)
    y = lax.bitcast_convert_type(i, jnp.float32)
    for _ in range(2):
        y = y * (1.5 - 0.5 * x * y * y)
    return y


def _sc_body(nc, ids_hbm, tags_hbm, table_hbm, comb_hbm, gam_hbm, bet_hbm,
             out_hbm, idx_v, tag_v, rows_v, comb_v, gam_v, bet_v, tbuf_v,
             sem_i, sem_g, sem_w):
    wid = lax.axis_index("s") * nc + lax.axis_index("c")
    per_w = N // (nc * 16)
    base = wid * per_w
    n_chunks = per_w // CHUNK

    pltpu.sync_copy(comb_hbm, comb_v)
    pltpu.sync_copy(gam_hbm, gam_v)
    pltpu.sync_copy(bet_hbm, bet_v)

    gvs = [gam_v[pl.ds(k * LANES, LANES)] for k in range(D // LANES)]
    bvs = [bet_v[pl.ds(k * LANES, LANES)] for k in range(D // LANES)]

    lanes = lax.iota(jnp.int32, LANES)
    lanes64 = lanes * D

    def issue_idx(ci, par):
        off = base + ci * CHUNK
        pltpu.async_copy(ids_hbm.at[pl.ds(off, CHUNK)], idx_v[par], sem_i[par])
        pltpu.async_copy(tags_hbm.at[pl.ds(off, CHUNK)], tag_v[par], sem_i[par])

    def wait_idx(par):
        pltpu.make_async_copy(ids_hbm.at[pl.ds(0, CHUNK)], idx_v[par], sem_i[par]).wait()
        pltpu.make_async_copy(tags_hbm.at[pl.ds(0, CHUNK)], tag_v[par], sem_i[par]).wait()

    def issue_gather(par):
        pltpu.async_copy(table_hbm.at[idx_v[par]], rows_v[par], sem_g[par])

    def wait_gather(par):
        pltpu.make_async_copy(table_hbm.at[idx_v[par]], rows_v[par], sem_g[par]).wait()

    def issue_wb(ci, par):
        off = base + ci * CHUNK
        pltpu.async_copy(rows_v[par], out_hbm.at[pl.ds(off, CHUNK)], sem_w[par])

    def wait_wb(par):
        pltpu.make_async_copy(rows_v[par], out_hbm.at[pl.ds(0, CHUNK)], sem_w[par]).wait()

    def bcast(v, j):
        return v.at[jnp.full((LANES,), j, jnp.int32)].get(mode="promise_in_bounds")

    def compute(ci, par):
        off = base + ci * CHUNK
        rows = rows_v[par]
        tags = tag_v[par]

        @pl.loop(0, CHUNK // LANES)
        def _group(g):
            tok = g * LANES + lanes
            gpos = off + tok
            lpos = lax.rem(gpos, jnp.int32(L))
            tag16 = tags[pl.ds(g * LANES, LANES)]
            cflat = (tag16 * L + lpos) * D

            PRE = 4
            vt = [None] * D
            vc = [None] * D

            def _issue(d):
                dvec = jnp.full((LANES,), d, jnp.int32)
                vt[d] = plsc.load_gather(rows, [tok, dvec])
                vc[d] = plsc.load_gather(comb_v, [cflat + d])

            for d in range(PRE):
                _issue(d)
            acc = jnp.zeros((LANES,), jnp.float32)
            acc2 = jnp.zeros((LANES,), jnp.float32)
            for d in range(D):
                if d + PRE < D:
                    _issue(d + PRE)
                v = vt[d] + vc[d]
                plsc.store_scatter(tbuf_v, [lanes64 + d], v)
                acc = acc + v
                acc2 = acc2 + v * v

            mean = acc * (1.0 / D)
            var = acc2 * (1.0 / D) - mean * mean
            rstd = _rsqrt_sc(var + EPS)
            mrs = mean * rstd

            KD = D // LANES
            ybuf = [None] * LANES

            def _ld_tok(t):
                ybuf[t] = [
                    tbuf_v[pl.ds(t * D + k * LANES, LANES)] for k in range(KD)
                ]

            def _fin_tok(t):
                a_t = bcast(rstd, t)
                m_t = bcast(mrs, t)
                y = ybuf[t]
                p = [y[k] * a_t for k in range(KD)]
                q = [p[k] - m_t for k in range(KD)]
                r = [q[k] * gvs[k] for k in range(KD)]
                s = [r[k] + bvs[k] for k in range(KD)]
                for k in range(KD):
                    rows[g * LANES + t, pl.ds(k * LANES, LANES)] = s[k]

            _ld_tok(0)
            for t in range(LANES):
                if t + 1 < LANES:
                    _ld_tok(t + 1)
                _fin_tok(t)

    issue_idx(0, 0)
    issue_idx(1, 1)
    wait_idx(0)
    issue_gather(0)

    @pl.loop(0, n_chunks // 2)
    def _super(sc_i):
        for par in range(2):
            ci = sc_i * 2 + par
            wait_gather(par)

            @pl.when(ci < n_chunks - 1)
            def _():
                wait_idx(1 - par)

                @pl.when(ci >= 1)
                def _():
                    wait_wb(1 - par)

                issue_gather(1 - par)

            compute(ci, par)
            issue_wb(ci, par)

            @pl.when(ci < n_chunks - 2)
            def _():
                issue_idx(ci + 2, par)

    wait_wb(0)
    wait_wb(1)


def kernel(input_ids, is_buyer_tags, token_table, buyer_table, ln_gamma, ln_beta):
    info = plsc.get_sparse_core_info()
    nc = info.num_cores

    ids_flat = input_ids.reshape(N).astype(jnp.int32)
    tags_flat = is_buyer_tags.reshape(N).astype(jnp.int32)
    pe = _positional_encoding(D, L)
    comb = (pe[None, :, :] + buyer_table[:, None, :]).reshape(2 * L * D)

    mesh = plsc.VectorSubcoreMesh(core_axis_name="c", subcore_axis_name="s")
    run = pl.kernel(
        functools.partial(_sc_body, nc),
        out_type=jax.ShapeDtypeStruct((N, D), jnp.float32),
        mesh=mesh,
        scratch_types=[
            [pltpu.VMEM((CHUNK,), jnp.int32)] * 2,
            [pltpu.VMEM((CHUNK,), jnp.int32)] * 2,
            [pltpu.VMEM((CHUNK, D), jnp.float32)] * 2,
            pltpu.VMEM((2 * L * D,), jnp.float32),
            pltpu.VMEM((D,), jnp.float32),
            pltpu.VMEM((D,), jnp.float32),
            pltpu.VMEM((D * LANES,), jnp.float32),
            [pltpu.SemaphoreType.DMA] * 2,
            [pltpu.SemaphoreType.DMA] * 2,
            [pltpu.SemaphoreType.DMA] * 2,
        ],
        compiler_params=pltpu.CompilerParams(
            use_tc_tiling_on_sc=False, needs_layout_passes=False
        ),
    )
    out = run(ids_flat, tags_flat, token_table, comb, ln_gamma, ln_beta)
    return out.reshape(B, L, D)


# speedup vs baseline: 2.9387x; 1.8125x over previous
import functools

import jax
import jax.numpy as jnp
from jax import lax
from jax.experimental import pallas as pl
from jax.experimental.pallas import tpu as pltpu
from jax.experimental.pallas import tpu_sc as plsc

VOCAB = 1000000
D = 64
B = 4096
L = 200
N = B * L
LANES = 16
CHUNK = 512
EPS = 1e-5


def _positional_encoding(d_model, max_len):
    pos = jnp.arange(max_len, dtype=jnp.float32)[:, None]
    div = jnp.exp(
        jnp.arange(0, d_model, 2, dtype=jnp.float32) * (-jnp.log(10000.0) / d_model)
    )
    ang = pos * div[None, :]
    pe = jnp.zeros((max_len, d_model), dtype=jnp.float32)
    pe = pe.at[:, 0::2].set(jnp.sin(ang))
    pe = pe.at[:, 1::2].set(jnp.cos(ang))
    return pe


def _rsqrt_sc(x):
    i = lax.bitcast_convert_type(x, jnp.int32)
    i = jnp.int32(0x5F3759DF) - (i >> 1)
    y = lax.bitcast_convert_type(i, jnp.float32)
    for _ in range(2):
        y = y * (1.5 - 0.5 * x * y * y)
    return y


def _sc_body(nc, ids_hbm, tags_hbm, table_hbm, comb_hbm, gam_hbm, bet_hbm,
             out_hbm, idx_v, tag_v, rows_v, comb_v, gam_v, bet_v, tbuf_v,
             sem_i, sem_g, sem_w):
    wid = lax.axis_index("s") * nc + lax.axis_index("c")
    per_w = N // (nc * 16)
    base = wid * per_w
    n_chunks = per_w // CHUNK

    pltpu.sync_copy(comb_hbm, comb_v)
    pltpu.sync_copy(gam_hbm, gam_v)
    pltpu.sync_copy(bet_hbm, bet_v)

    gvs = [gam_v[pl.ds(k * LANES, LANES)] for k in range(D // LANES)]
    bvs = [bet_v[pl.ds(k * LANES, LANES)] for k in range(D // LANES)]

    lanes = lax.iota(jnp.int32, LANES)
    lanes64 = lanes * D

    def issue_idx(ci, par):
        off = base + ci * CHUNK
        pltpu.async_copy(ids_hbm.at[pl.ds(off, CHUNK)], idx_v[par], sem_i[par])
        pltpu.async_copy(tags_hbm.at[pl.ds(off, CHUNK)], tag_v[par], sem_i[par])

    def wait_idx(par):
        pltpu.make_async_copy(ids_hbm.at[pl.ds(0, CHUNK)], idx_v[par], sem_i[par]).wait()
        pltpu.make_async_copy(tags_hbm.at[pl.ds(0, CHUNK)], tag_v[par], sem_i[par]).wait()

    def issue_gather(par):
        pltpu.async_copy(table_hbm.at[idx_v[par]], rows_v[par], sem_g[par])

    def wait_gather(par):
        pltpu.make_async_copy(table_hbm.at[idx_v[par]], rows_v[par], sem_g[par]).wait()

    def issue_wb(ci, par):
        off = base + ci * CHUNK
        pltpu.async_copy(rows_v[par], out_hbm.at[pl.ds(off, CHUNK)], sem_w[par])

    def wait_wb(par):
        pltpu.make_async_copy(rows_v[par], out_hbm.at[pl.ds(0, CHUNK)], sem_w[par]).wait()

    def bcast(v, j):
        return v.at[jnp.full((LANES,), j, jnp.int32)].get(mode="promise_in_bounds")

    def compute(ci, par):
        off = base + ci * CHUNK
        rows = rows_v[par]
        tags = tag_v[par]

        @pl.loop(0, CHUNK // LANES)
        def _group(g):
            tok = g * LANES + lanes
            gpos = off + tok
            lpos = lax.rem(gpos, jnp.int32(L))
            tag16 = tags[pl.ds(g * LANES, LANES)]
            cflat = (tag16 * L + lpos) * D

            PRE = 4
            vt = [None] * D
            vc = [None] * D
            rots = [None] * D

            def _issue(d):
                rot = lanes + d
                if d + LANES > D:
                    rot = rot & (D - 1)
                rots[d] = rot
                vt[d] = plsc.load_gather(rows, [tok, rot])
                vc[d] = plsc.load_gather(comb_v, [cflat + rot])

            for d in range(PRE):
                _issue(d)
            acc = jnp.zeros((LANES,), jnp.float32)
            acc2 = jnp.zeros((LANES,), jnp.float32)
            for d in range(D):
                if d + PRE < D:
                    _issue(d + PRE)
                v = vt[d] + vc[d]
                plsc.store_scatter(tbuf_v, [lanes64 + rots[d]], v)
                acc = acc + v
                acc2 = acc2 + v * v

            mean = acc * (1.0 / D)
            var = acc2 * (1.0 / D) - mean * mean
            rstd = _rsqrt_sc(var + EPS)
            mrs = mean * rstd

            KD = D // LANES
            ybuf = [None] * LANES

            def _ld_tok(t):
                ybuf[t] = [
                    tbuf_v[pl.ds(t * D + k * LANES, LANES)] for k in range(KD)
                ]

            def _fin_tok(t):
                a_t = bcast(rstd, t)
                m_t = bcast(mrs, t)
                y = ybuf[t]
                p = [y[k] * a_t for k in range(KD)]
                q = [p[k] - m_t for k in range(KD)]
                r = [q[k] * gvs[k] for k in range(KD)]
                s = [r[k] + bvs[k] for k in range(KD)]
                for k in range(KD):
                    rows[g * LANES + t, pl.ds(k * LANES, LANES)] = s[k]

            _ld_tok(0)
            for t in range(LANES):
                if t + 1 < LANES:
                    _ld_tok(t + 1)
                _fin_tok(t)

    issue_idx(0, 0)
    issue_idx(1, 1)
    wait_idx(0)
    issue_gather(0)

    @pl.loop(0, n_chunks // 2)
    def _super(sc_i):
        for par in range(2):
            ci = sc_i * 2 + par
            wait_gather(par)

            @pl.when(ci < n_chunks - 1)
            def _():
                wait_idx(1 - par)

                @pl.when(ci >= 1)
                def _():
                    wait_wb(1 - par)

                issue_gather(1 - par)

            compute(ci, par)
            issue_wb(ci, par)

            @pl.when(ci < n_chunks - 2)
            def _():
                issue_idx(ci + 2, par)

    wait_wb(0)
    wait_wb(1)


def kernel(input_ids, is_buyer_tags, token_table, buyer_table, ln_gamma, ln_beta):
    info = plsc.get_sparse_core_info()
    nc = info.num_cores

    ids_flat = input_ids.reshape(N).astype(jnp.int32)
    tags_flat = is_buyer_tags.reshape(N).astype(jnp.int32)
    pe = _positional_encoding(D, L)
    comb = (pe[None, :, :] + buyer_table[:, None, :]).reshape(2 * L * D)

    mesh = plsc.VectorSubcoreMesh(core_axis_name="c", subcore_axis_name="s")
    run = pl.kernel(
        functools.partial(_sc_body, nc),
        out_type=jax.ShapeDtypeStruct((N, D), jnp.float32),
        mesh=mesh,
        scratch_types=[
            [pltpu.VMEM((CHUNK,), jnp.int32)] * 2,
            [pltpu.VMEM((CHUNK,), jnp.int32)] * 2,
            [pltpu.VMEM((CHUNK, D), jnp.float32)] * 2,
            pltpu.VMEM((2 * L * D,), jnp.float32),
            pltpu.VMEM((D,), jnp.float32),
            pltpu.VMEM((D,), jnp.float32),
            pltpu.VMEM((D * LANES,), jnp.float32),
            [pltpu.SemaphoreType.DMA] * 2,
            [pltpu.SemaphoreType.DMA] * 2,
            [pltpu.SemaphoreType.DMA] * 2,
        ],
        compiler_params=pltpu.CompilerParams(
            use_tc_tiling_on_sc=False, needs_layout_passes=False
        ),
    )
    out = run(ids_flat, tags_flat, token_table, comb, ln_gamma, ln_beta)
    return out.reshape(B, L, D)
